# baseline (speedup 1.0000x reference)
import jax
import jax.numpy as jnp
from jax import lax
from jax.experimental import pallas as pl
from jax.experimental.pallas import tpu as pltpu
from jax.experimental.pallas import tpu_sc as plsc

BATCH = 16384
NFIELDS = 26
VOCAB = 1000000
NDENSE = 13

NC = 2
NS = 16
L = 16
NW = NC * NS
BPW = BATCH // NW
CHUNKS = BPW // L

FLO = 12
FHI = NFIELDS - FLO
NLO = FLO * BPW
NHI = FHI * BPW

A1_LEN = 11719 * 1024
A2_START = 11_999_872
A2_LEN = NFIELDS * VOCAB - A2_START


def _sc_body_hi(ta_hbm, idx_hbm, dense_hbm, w_hbm, part_hbm,
                idx_v, emb_v, dense_v, w_v, out_v, sem):
    wid = lax.axis_index("s") * NC + lax.axis_index("c")

    pltpu.sync_copy(idx_hbm.at[wid], idx_v)
    pltpu.sync_copy(dense_hbm.at[wid], dense_v)
    pltpu.sync_copy(w_hbm, w_v)

    def add_off_hi(c, _):
        for f in range(FHI):
            off = (FLO + f) * VOCAB - A2_START
            sl = pl.ds(f * BPW + c * L, L)
            idx_v[sl] = idx_v[sl] + off
        return 0

    lax.fori_loop(0, CHUNKS, add_off_hi, 0)

    g = pltpu.async_copy(ta_hbm.at[idx_v], emb_v, sem)

    wsplats = [w_v[j, :] for j in range(NDENSE)]

    def fc_chunk(c, _):
        sl = pl.ds(c * L, L)
        acc = dense_v[0, sl] * wsplats[0]
        for j in range(1, NDENSE):
            acc = acc + dense_v[j, sl] * wsplats[j]
        out_v[sl] = acc
        return 0

    lax.fori_loop(0, CHUNKS, fc_chunk, 0)
    g.wait()

    def red_chunk(c, _):
        sl = pl.ds(c * L, L)
        acc = out_v[sl]
        for f in range(FHI):
            acc = acc + emb_v[pl.ds(f * BPW + c * L, L)]
        out_v[sl] = acc
        return 0

    lax.fori_loop(0, CHUNKS, red_chunk, 0)
    pltpu.sync_copy(out_v, part_hbm.at[pl.ds(wid * BPW, BPW)])


def _sc_body_lo(ta_hbm, idx_hbm, part_hbm, out_hbm,
                idx_v, emb_v, out_v, sem):
    wid = lax.axis_index("s") * NC + lax.axis_index("c")

    pltpu.sync_copy(idx_hbm.at[wid], idx_v)

    def add_off_lo(c, _):
        for f in range(1, FLO):
            off = f * VOCAB
            sl = pl.ds(f * BPW + c * L, L)
            idx_v[sl] = idx_v[sl] + off
        return 0

    lax.fori_loop(0, CHUNKS, add_off_lo, 0)

    g = pltpu.async_copy(ta_hbm.at[idx_v], emb_v, sem)

    pltpu.sync_copy(part_hbm.at[pl.ds(wid * BPW, BPW)], out_v)
    g.wait()

    def red_chunk(c, _):
        sl = pl.ds(c * L, L)
        acc = out_v[sl]
        for f in range(FLO):
            acc = acc + emb_v[pl.ds(f * BPW + c * L, L)]
        out_v[sl] = acc
        return 0

    lax.fori_loop(0, CHUNKS, red_chunk, 0)
    pltpu.sync_copy(out_v, out_hbm.at[pl.ds(wid * BPW, BPW)])


@jax.jit
def _run(table_a1, table_a2, idx_lo, idx_hi, dense_rs, w_rep):
    mesh = plsc.VectorSubcoreMesh(core_axis_name="c", subcore_axis_name="s")
    k_hi = pl.kernel(
        _sc_body_hi,
        out_type=jax.ShapeDtypeStruct((BATCH,), jnp.float32),
        mesh=mesh,
        scratch_types=[
            pltpu.VMEM((NHI,), jnp.int32),
            pltpu.VMEM((NHI,), jnp.float32),
            pltpu.VMEM((NDENSE, BPW), jnp.float32),
            pltpu.VMEM((NDENSE, L), jnp.float32),
            pltpu.VMEM((BPW,), jnp.float32),
            pltpu.SemaphoreType.DMA,
        ],
    )
    k_lo = pl.kernel(
        _sc_body_lo,
        out_type=jax.ShapeDtypeStruct((BATCH,), jnp.float32),
        mesh=mesh,
        scratch_types=[
            pltpu.VMEM((NLO,), jnp.int32),
            pltpu.VMEM((NLO,), jnp.float32),
            pltpu.VMEM((BPW,), jnp.float32),
            pltpu.SemaphoreType.DMA,
        ],
    )
    partial = k_hi(table_a2, idx_hi, dense_rs, w_rep)
    return k_lo(table_a1, idx_lo, partial)


def kernel(indices, dense, emb_table, W):
    i3 = indices.T.reshape(NFIELDS, NW, BPW).transpose(1, 0, 2)
    idx_lo = i3[:, :FLO, :].reshape(NW, NLO)
    idx_hi = i3[:, FLO:, :].reshape(NW, NHI)
    dense_rs = dense.T.reshape(NDENSE, NW, BPW).transpose(1, 0, 2)
    table_a1 = emb_table[:A1_LEN, :].reshape(-1)
    table_a2 = emb_table[A2_START:, :].reshape(-1)
    w_rep = jnp.broadcast_to(W, (NDENSE, L))
    out = _run(table_a1, table_a2, idx_lo, idx_hi, dense_rs, w_rep)
    return out.reshape(-1, 1)


# speedup vs baseline: 1.0082x; 1.0082x over previous
import jax
import jax.numpy as jnp
from jax import lax
from jax.experimental import pallas as pl
from jax.experimental.pallas import tpu as pltpu
from jax.experimental.pallas import tpu_sc as plsc

BATCH = 16384
NFIELDS = 26
VOCAB = 1000000
NDENSE = 13

NC = 2
NS = 16
L = 16
NW = NC * NS
BPW = BATCH // NW
CHUNKS = BPW // L

FLO = 10
FHI = NFIELDS - FLO
NLO = FLO * BPW
NHI = FHI * BPW

A1_LEN = 9766 * 1024
A2_START = 10_000_000
A2_LEN = NFIELDS * VOCAB - A2_START


def _sc_body_hi(ta_hbm, idx_hbm, dense_hbm, w_hbm, part_hbm,
                idx_v, emb_v, dense_v, w_v, out_v, sem):
    wid = lax.axis_index("s") * NC + lax.axis_index("c")

    pltpu.sync_copy(idx_hbm.at[wid], idx_v)
    pltpu.sync_copy(dense_hbm.at[wid], dense_v)
    pltpu.sync_copy(w_hbm, w_v)

    def add_off_hi(c, _):
        for f in range(FHI):
            off = (FLO + f) * VOCAB - A2_START
            sl = pl.ds(f * BPW + c * L, L)
            idx_v[sl] = idx_v[sl] + off
        return 0

    lax.fori_loop(0, CHUNKS, add_off_hi, 0)

    g = pltpu.async_copy(ta_hbm.at[idx_v], emb_v, sem)

    wsplats = [w_v[j, :] for j in range(NDENSE)]

    def fc_chunk(c, _):
        sl = pl.ds(c * L, L)
        acc = dense_v[0, sl] * wsplats[0]
        for j in range(1, NDENSE):
            acc = acc + dense_v[j, sl] * wsplats[j]
        out_v[sl] = acc
        return 0

    lax.fori_loop(0, CHUNKS, fc_chunk, 0)
    g.wait()

    def red_chunk(c, _):
        sl = pl.ds(c * L, L)
        acc = out_v[sl]
        for f in range(FHI):
            acc = acc + emb_v[pl.ds(f * BPW + c * L, L)]
        out_v[sl] = acc
        return 0

    lax.fori_loop(0, CHUNKS, red_chunk, 0)
    pltpu.sync_copy(out_v, part_hbm.at[pl.ds(wid * BPW, BPW)])


def _sc_body_lo(ta_hbm, idx_hbm, part_hbm, out_hbm,
                idx_v, emb_v, out_v, sem):
    wid = lax.axis_index("s") * NC + lax.axis_index("c")

    pltpu.sync_copy(idx_hbm.at[wid], idx_v)

    def add_off_lo(c, _):
        for f in range(1, FLO):
            off = f * VOCAB
            sl = pl.ds(f * BPW + c * L, L)
            idx_v[sl] = idx_v[sl] + off
        return 0

    lax.fori_loop(0, CHUNKS, add_off_lo, 0)

    g = pltpu.async_copy(ta_hbm.at[idx_v], emb_v, sem)

    pltpu.sync_copy(part_hbm.at[pl.ds(wid * BPW, BPW)], out_v)
    g.wait()

    def red_chunk(c, _):
        sl = pl.ds(c * L, L)
        acc = out_v[sl]
        for f in range(FLO):
            acc = acc + emb_v[pl.ds(f * BPW + c * L, L)]
        out_v[sl] = acc
        return 0

    lax.fori_loop(0, CHUNKS, red_chunk, 0)
    pltpu.sync_copy(out_v, out_hbm.at[pl.ds(wid * BPW, BPW)])


@jax.jit
def _run(table_a1, table_a2, idx_lo, idx_hi, dense_rs, w_rep):
    mesh = plsc.VectorSubcoreMesh(core_axis_name="c", subcore_axis_name="s")
    k_hi = pl.kernel(
        _sc_body_hi,
        out_type=jax.ShapeDtypeStruct((BATCH,), jnp.float32),
        mesh=mesh,
        scratch_types=[
            pltpu.VMEM((NHI,), jnp.int32),
            pltpu.VMEM((NHI,), jnp.float32),
            pltpu.VMEM((NDENSE, BPW), jnp.float32),
            pltpu.VMEM((NDENSE, L), jnp.float32),
            pltpu.VMEM((BPW,), jnp.float32),
            pltpu.SemaphoreType.DMA,
        ],
    )
    k_lo = pl.kernel(
        _sc_body_lo,
        out_type=jax.ShapeDtypeStruct((BATCH,), jnp.float32),
        mesh=mesh,
        scratch_types=[
            pltpu.VMEM((NLO,), jnp.int32),
            pltpu.VMEM((NLO,), jnp.float32),
            pltpu.VMEM((BPW,), jnp.float32),
            pltpu.SemaphoreType.DMA,
        ],
    )
    partial = k_hi(table_a2, idx_hi, dense_rs, w_rep)
    return k_lo(table_a1, idx_lo, partial)


def kernel(indices, dense, emb_table, W):
    i3 = indices.T.reshape(NFIELDS, NW, BPW).transpose(1, 0, 2)
    idx_lo = i3[:, :FLO, :].reshape(NW, NLO)
    idx_hi = i3[:, FLO:, :].reshape(NW, NHI)
    dense_rs = dense.T.reshape(NDENSE, NW, BPW).transpose(1, 0, 2)
    table_a1 = emb_table[:A1_LEN, :].reshape(-1)
    table_a2 = emb_table[A2_START:, :].reshape(-1)
    w_rep = jnp.broadcast_to(W, (NDENSE, L))
    out = _run(table_a1, table_a2, idx_lo, idx_hi, dense_rs, w_rep)
    return out.reshape(-1, 1)
